# Initial kernel scaffold; baseline (speedup 1.0000x reference)
import jax
import jax.numpy as jnp
from jax.experimental import pallas as pl


def kernel(loss, x, t, beta):
    raise NotImplementedError("write your pallas kernel here")


# speedup vs baseline: 6.9215x; 6.9215x over previous
import functools

import jax
import jax.numpy as jnp
from jax import lax
from jax.experimental import pallas as pl
from jax.experimental.pallas import tpu as pltpu
from jax.experimental.pallas import tpu_sc as plsc

N = 1000000
X_LIM = (-1.0, 1.0)
T_LIM = (0.0, 1.0)
ALPHA = 10.0

NP = 1 << 20
NC, NS, L = 2, 16, 16
NW = NC * NS
CH = NP // NW
ROWS = CH // 128

_mesh = plsc.VectorSubcoreMesh(
    core_axis_name="c", subcore_axis_name="s", num_cores=NC, num_subcores=NS)


def _wid():
    return lax.axis_index("s") * NC + lax.axis_index("c")


@functools.partial(
    pl.kernel,
    compiler_params=pltpu.CompilerParams(needs_layout_passes=False),
    out_type=(
        jax.ShapeDtypeStruct((NW, L), jnp.int32),
        jax.ShapeDtypeStruct((2 * NP,), jnp.float32),
        jax.ShapeDtypeStruct((2 * NP,), jnp.float32),
    ),
    mesh=_mesh,
    scratch_types=(
        pltpu.VMEM((2048,), jnp.float32),
        pltpu.VMEM((2048,), jnp.float32),
        pltpu.VMEM((2048,), jnp.float32),
        pltpu.VMEM((CH + 16,), jnp.float32),
        pltpu.VMEM((CH + 16,), jnp.float32),
        pltpu.VMEM((L,), jnp.float32),
        pltpu.VMEM((L,), jnp.int32),
    ),
)
def _k1(fit_hbm, x_hbm, t_hbm, xn_hbm, tn_hbm, mean_hbm,
        counts_hbm, cxx_hbm, ctt_hbm,
        fbuf, xbuf, tbuf, bufx, buft, meanv, cntv):
    w = _wid()
    base = w * CH
    pltpu.sync_copy(xn_hbm.at[pl.ds(base, CH)], cxx_hbm.at[pl.ds(NP + base, CH)])
    pltpu.sync_copy(tn_hbm.at[pl.ds(base, CH)], ctt_hbm.at[pl.ds(NP + base, CH)])
    pltpu.sync_copy(mean_hbm, meanv)
    mean = meanv[...]

    SB = 2048

    def outer(sb, off):
        s0 = base + sb * SB
        pltpu.sync_copy(fit_hbm.at[pl.ds(s0, SB)], fbuf)
        pltpu.sync_copy(x_hbm.at[pl.ds(s0, SB)], xbuf)
        pltpu.sync_copy(t_hbm.at[pl.ds(s0, SB)], tbuf)

        def inner(i, off):
            o = i * L
            m = fbuf[pl.ds(o, L)] > mean
            mi = m.astype(jnp.int32)
            pos = plsc.cumsum(mi) - mi + off
            plsc.store_scatter(bufx, [pos], xbuf[pl.ds(o, L)], mask=m)
            plsc.store_scatter(buft, [pos], tbuf[pl.ds(o, L)], mask=m)
            return off + jnp.sum(mi)

        return lax.fori_loop(0, SB // L, inner, off)

    cnt = lax.fori_loop(0, CH // SB, outer, jnp.int32(0))
    pltpu.sync_copy(bufx.at[pl.ds(0, CH)], cxx_hbm.at[pl.ds(base, CH)])
    pltpu.sync_copy(buft.at[pl.ds(0, CH)], ctt_hbm.at[pl.ds(base, CH)])
    cntv[...] = jnp.broadcast_to(cnt, (L,))
    pltpu.sync_copy(cntv, counts_hbm.at[w])


@functools.partial(
    pl.kernel,
    compiler_params=pltpu.CompilerParams(needs_layout_passes=False),
    out_type=(
        jax.ShapeDtypeStruct((NW, CH), jnp.float32),
        jax.ShapeDtypeStruct((NW, CH), jnp.float32),
    ),
    mesh=_mesh,
    scratch_types=(
        pltpu.VMEM((NW, L), jnp.int32),
        pltpu.VMEM((48,), jnp.int32),
        pltpu.SMEM((40,), jnp.int32),
        pltpu.SMEM((40,), jnp.int32),
        pltpu.VMEM((CH,), jnp.int32),
        pltpu.VMEM((CH,), jnp.float32),
        pltpu.VMEM((CH,), jnp.float32),
        pltpu.SemaphoreType.DMA,
        pltpu.SemaphoreType.DMA,
    ),
)
def _k2(counts_hbm, cxx_hbm, ctt_hbm, xo_hbm, to_hbm,
        cntbuf, shiftv, bound_s, shift_s, idxbuf, gbx, gbt, sem1, sem2):
    w = _wid()
    obase = w * CH
    pltpu.sync_copy(counts_hbm, cntbuf)

    lane0 = lax.iota(jnp.int32, L) == 0
    b = jnp.int32(0)
    bound_s[0] = b
    for c in range(NW):
        sh = jnp.int32(c * CH) - b
        shift_s[c] = sh
        plsc.store_scatter(shiftv, [jnp.full((L,), c, jnp.int32)],
                           jnp.broadcast_to(sh, (L,)), mask=lane0)
        b = b + jnp.max(cntbuf[c])
        bound_s[c + 1] = b
    k_total = b
    sh_tail = jnp.int32(NP) - k_total
    shift_s[NW] = sh_tail
    plsc.store_scatter(shiftv, [jnp.full((L,), NW, jnp.int32)],
                       jnp.broadcast_to(sh_tail, (L,)), mask=lane0)
    bound_s[NW + 1] = jnp.int32(NP)

    iota16 = lax.iota(jnp.int32, L)

    def slow_lane_idx(r):
        p = jnp.zeros((L,), jnp.int32)
        for q in range(1, NW + 1):
            p = p + (r >= bound_s[q]).astype(jnp.int32)
        shl = plsc.load_gather(shiftv, [p])
        return r + shl

    def row(i, p):
        j = obase + i * 128

        def adv_cond(p):
            return jnp.logical_and(p < NW, j >= bound_s[p + 1])

        p = lax.while_loop(adv_cond, lambda p: p + 1, p)
        hi = bound_s[p + 1]
        sh = shift_s[p]

        def fast(_):
            for l in range(8):
                r = j + l * L + iota16
                idxbuf[pl.ds(i * 128 + l * L, L)] = r + sh
            return 0

        def slow(_):
            for l in range(8):
                r = j + l * L + iota16
                idxbuf[pl.ds(i * 128 + l * L, L)] = slow_lane_idx(r)
            return 0

        lax.cond(j + 127 < hi, fast, slow, 0)
        return p

    lax.fori_loop(0, ROWS, row, jnp.int32(0))

    cpx = pltpu.async_copy(cxx_hbm.at[idxbuf], gbx, sem1)
    cpt = pltpu.async_copy(ctt_hbm.at[idxbuf], gbt, sem2)
    cpx.wait()
    cpt.wait()
    pltpu.sync_copy(gbx, xo_hbm.at[w])
    pltpu.sync_copy(gbt, to_hbm.at[w])


def kernel(loss, x, t, beta):
    f32 = jnp.float32
    t_norm = (t - T_LIM[0]) / (T_LIM[1] - T_LIM[0])
    gate = jax.nn.relu(-jnp.tanh(ALPHA * (t_norm - beta)))
    fitness = loss * gate
    mean = fitness.mean()

    kr = jax.random.key(1)
    ka, kb = jax.random.split(kr)
    x_new = jax.random.uniform(ka, (N, 1), dtype=f32, minval=X_LIM[0], maxval=X_LIM[1])
    t_new = jax.random.uniform(kb, (N, 1), dtype=f32, minval=T_LIM[0], maxval=T_LIM[1])

    pad = NP - N
    fit_p = jnp.concatenate([fitness[:, 0], jnp.full((pad,), -1.0, f32)])
    x_p = jnp.concatenate([x[:, 0], jnp.zeros((pad,), f32)])
    t_p = jnp.concatenate([t[:, 0], jnp.zeros((pad,), f32)])
    xn_p = jnp.concatenate([x_new[:, 0], jnp.zeros((pad,), f32)])
    tn_p = jnp.concatenate([t_new[:, 0], jnp.zeros((pad,), f32)])
    mean_v = jnp.broadcast_to(mean, (L,))

    counts, cxx, ctt = _k1(fit_p, x_p, t_p, xn_p, tn_p, mean_v)
    xo, to = _k2(counts, cxx, ctt)
    x_out = xo.reshape(NP, 1)[:N]
    t_out = to.reshape(NP, 1)[:N]
    return (x_out, t_out)
